# Initial kernel scaffold; baseline (speedup 1.0000x reference)
import jax
import jax.numpy as jnp
from jax.experimental import pallas as pl


def kernel(x_tx, x_addr, edge_t2a, edge_a2t, batch_tx, proj_tx_W, proj_tx_b, proj_addr_W, proj_addr_b, han_tx_W, han_tx_b, han_addr_W, han_addr_b, att_src_t2a, att_dst_t2a, att_src_a2t, att_dst_a2t, q_sem, k_lin_W, k_lin_b, cls1_W, cls1_b, cls2_W, cls2_b):
    raise NotImplementedError("write your pallas kernel here")


# speedup vs baseline: 6.8666x; 6.8666x over previous
import functools

import jax
import jax.numpy as jnp
from jax import lax
from jax.experimental import pallas as pl
from jax.experimental.pallas import tpu as pltpu
from jax.experimental.pallas import tpu_sc as plsc

N_NODE = 50000
E_EDGES = 800000
H_HEADS = 4
D_HEAD = 16
HID = 64
NG = 256

NC = 2
NS = 16
NW = NC * NS
CHUNK = 128
GROUP = 1024
E_PAD = 819200
N_PAD = 50176
GROUPS_PER_TILE = E_PAD // (NW * GROUP)
ZERO_ITERS = N_PAD // GROUP

_BLK = 2000
_NBLK = N_NODE // _BLK


def _pre_body(xt_ref, xa_ref, wt_ref, bt_ref, wa_ref, ba_ref, asrc_ref,
              adst_ref, z0_ref, z1_ref, msrc_ref, mdst_ref):
    zt = jnp.dot(xt_ref[...], wt_ref[...].T,
                 preferred_element_type=jnp.float32) + bt_ref[...]
    za = jnp.dot(xa_ref[...], wa_ref[...].T,
                 preferred_element_type=jnp.float32) + ba_ref[...]
    z0_ref[0] = za[:, 0:16]
    z0_ref[1] = za[:, 16:32]
    z1_ref[0] = za[:, 32:48]
    z1_ref[1] = za[:, 48:64]
    pad = jnp.zeros((za.shape[0], 12), jnp.float32)
    a_s = jnp.dot(za, asrc_ref[...], preferred_element_type=jnp.float32)
    a_d = jnp.dot(zt, adst_ref[...], preferred_element_type=jnp.float32)
    msrc_ref[...] = jnp.concatenate([a_s, pad], axis=1)
    mdst_ref[...] = jnp.concatenate([a_d, pad], axis=1)


def _run_pre(x_tx, x_addr, w_tx, b_tx, w_ad, b_ad, a_src_m, a_dst_m):
    f32 = jnp.float32
    row_spec = pl.BlockSpec((_BLK, 65), lambda i: (i, 0))
    full = lambda shape: pl.BlockSpec(shape, lambda i: tuple(0 for _ in shape))
    out16 = pl.BlockSpec((_BLK, 16), lambda i: (i, 0))
    zspec = pl.BlockSpec((2, _BLK, 16), lambda i: (0, i, 0))
    return pl.pallas_call(
        _pre_body,
        grid=(_NBLK,),
        in_specs=[row_spec, row_spec, full((64, 65)), full((1, 64)),
                  full((64, 65)), full((1, 64)), full((64, 4)), full((64, 4))],
        out_specs=[zspec, zspec, out16, out16],
        out_shape=[jax.ShapeDtypeStruct((2, N_NODE, 16), f32)] * 2
        + [jax.ShapeDtypeStruct((N_NODE, 16), f32),
           jax.ShapeDtypeStruct((N_PAD, 16), f32)],
    )(x_tx, x_addr, w_tx, b_tx, w_ad, b_ad, a_src_m, a_dst_m)


def _pass1_body(src_hbm, dst_hbm, msrc_hbm, mdst_hbm, e_hbm, ssum_hbm,
                srcv, dstv, msv, mdv, ev, ssum_sh):
    cid = lax.axis_index("c")
    sid = lax.axis_index("s")
    w = cid * NS + sid

    @pl.loop(0, GROUP)
    def _zero_ev(r):
        ev[pl.ds(r, 1), :] = jnp.zeros((1, 16), jnp.float32)

    @pl.loop(0, 4)
    def _zero_shared(g):
        idx = sid + g * NS

        @pl.when(idx < ZERO_ITERS)
        def _():
            pltpu.sync_copy(ev, ssum_sh.at[pl.ds(idx * GROUP, GROUP)])

    plsc.subcore_barrier()

    @pl.loop(0, GROUPS_PER_TILE)
    def _group(g):
        chunk0 = w * (GROUPS_PER_TILE * 8) + g * 8
        pltpu.sync_copy(src_hbm.at[pl.ds(chunk0, 8)], srcv)
        pltpu.sync_copy(dst_hbm.at[pl.ds(chunk0, 8)], dstv)
        for j in range(8):
            pltpu.sync_copy(msrc_hbm.at[srcv.at[j]],
                            msv.at[pl.ds(j * CHUNK, CHUNK)])
            pltpu.sync_copy(mdst_hbm.at[dstv.at[j]],
                            mdv.at[pl.ds(j * CHUNK, CHUNK)])

        @pl.loop(0, GROUP)
        def _row(r):
            s = msv[pl.ds(r, 1), :] + mdv[pl.ds(r, 1), :]
            ev[pl.ds(r, 1), :] = jnp.exp(jnp.maximum(s, 0.2 * s))

        for j in range(8):
            pltpu.sync_copy(ev.at[pl.ds(j * CHUNK, CHUNK)],
                            ssum_sh.at[dstv.at[j]], add=True)
        pltpu.sync_copy(ev, e_hbm.at[pl.ds((w * GROUPS_PER_TILE + g) * GROUP,
                                           GROUP)])

    plsc.subcore_barrier()

    @pl.loop(0, 4)
    def _writeout(g):
        idx = sid + g * NS

        @pl.when(idx < ZERO_ITERS)
        def _():
            pltpu.sync_copy(ssum_sh.at[pl.ds(idx * GROUP, GROUP)],
                            ssum_hbm.at[cid, pl.ds(idx * GROUP, GROUP)])


def _run_pass1(src2d, dst2d, msrc, mdst):
    f32 = jnp.float32
    mesh = plsc.VectorSubcoreMesh(core_axis_name="c", subcore_axis_name="s")
    return pl.kernel(
        _pass1_body,
        mesh=mesh,
        compiler_params=pltpu.CompilerParams(use_tc_tiling_on_sc=False),
        out_type=[jax.ShapeDtypeStruct((E_PAD, 16), f32),
                  jax.ShapeDtypeStruct((NC, N_PAD, 16), f32)],
        scratch_types=[pltpu.VMEM((8, CHUNK), jnp.int32),
                       pltpu.VMEM((8, CHUNK), jnp.int32),
                       pltpu.VMEM((GROUP, 16), f32),
                       pltpu.VMEM((GROUP, 16), f32),
                       pltpu.VMEM((GROUP, 16), f32),
                       pltpu.VMEM_SHARED((N_PAD, 16), f32)],
    )(src2d, dst2d, msrc, mdst)


P2_GROUPS = E_PAD // (NS * GROUP)


def _pass2_body(h0, h1, src_hbm, dst_hbm, e_hbm, z_hbm, acc_hbm,
                srcv, dstv, ev, zv, mv, acc_sh):
    cid = lax.axis_index("c")
    sid = lax.axis_index("s")

    @pl.loop(0, GROUP)
    def _zero_mv(r):
        mv[pl.ds(r, 1), :] = jnp.zeros((1, 16), jnp.float32)

    @pl.loop(0, 4)
    def _zero_shared(g):
        idx = sid + g * NS

        @pl.when(idx < ZERO_ITERS)
        def _():
            pltpu.sync_copy(mv, acc_sh.at[pl.ds(idx * GROUP, GROUP)])

    plsc.subcore_barrier()

    @pl.loop(0, P2_GROUPS)
    def _group(g):
        chunk0 = sid * (P2_GROUPS * 8) + g * 8
        base_e = (sid * P2_GROUPS + g) * GROUP
        pltpu.sync_copy(src_hbm.at[pl.ds(chunk0, 8)], srcv)
        pltpu.sync_copy(dst_hbm.at[pl.ds(chunk0, 8)], dstv)
        pltpu.sync_copy(e_hbm.at[pl.ds(base_e, GROUP)], ev)
        for j in range(8):
            pltpu.sync_copy(z_hbm.at[cid].at[srcv.at[j]],
                            zv.at[pl.ds(j * CHUNK, CHUNK)])

        @pl.loop(0, GROUP)
        def _row(r):
            e_row = ev[pl.ds(r, 1), :]
            ea = jnp.where(cid == 0, e_row[0, h0], e_row[0, h1])
            mv[pl.ds(r, 1), :] = zv[pl.ds(r, 1), :] * ea

        for j in range(8):
            pltpu.sync_copy(mv.at[pl.ds(j * CHUNK, CHUNK)],
                            acc_sh.at[dstv.at[j]], add=True)

    plsc.subcore_barrier()

    @pl.loop(0, 4)
    def _writeout(g):
        idx = sid + g * NS

        @pl.when(idx < ZERO_ITERS)
        def _():
            pltpu.sync_copy(acc_sh.at[pl.ds(idx * GROUP, GROUP)],
                            acc_hbm.at[cid, pl.ds(idx * GROUP, GROUP)])


def _run_pass2(h0, h1, src2d, dst2d, e_all, zpair):
    f32 = jnp.float32
    mesh = plsc.VectorSubcoreMesh(core_axis_name="c", subcore_axis_name="s")
    return pl.kernel(
        functools.partial(_pass2_body, h0, h1),
        mesh=mesh,
        compiler_params=pltpu.CompilerParams(use_tc_tiling_on_sc=False),
        out_type=jax.ShapeDtypeStruct((NC, N_PAD, 16), f32),
        scratch_types=[pltpu.VMEM((8, CHUNK), jnp.int32),
                       pltpu.VMEM((8, CHUNK), jnp.int32),
                       pltpu.VMEM((GROUP, 16), f32),
                       pltpu.VMEM((GROUP, 16), f32),
                       pltpu.VMEM((GROUP, 16), f32),
                       pltpu.VMEM_SHARED((N_PAD, 16), f32)],
    )(src2d, dst2d, e_all, zpair)


def _post_body(ssum_ref, acc01_ref, acc23_ref, bt_ref, w1_ref, b1_ref,
               w2_ref, b2_ref, out_ref, sums_ref):
    i = pl.program_id(0)
    ssum = ssum_ref[0] + ssum_ref[1]
    cols = []
    for pair, ref in ((0, acc01_ref), (1, acc23_ref)):
        for k in range(2):
            h = pair * 2 + k
            a = ref[k]
            denom = ssum[:, h:h + 1] + 1e-16
            cols.append(a / denom)
    out_blk = jnp.maximum(jnp.concatenate(cols, axis=1), 0.0)
    ids = bt_ref[0, 0, :]
    onehot = (ids[:, None] == lax.broadcasted_iota(jnp.int32, (_BLK, NG), 1))
    onehot = onehot.astype(jnp.float32)
    feat = jnp.concatenate(
        [out_blk, jnp.ones((_BLK, 1), jnp.float32)], axis=1)
    part = lax.dot_general(onehot, feat, (((0,), (0,)), ((), ())),
                           preferred_element_type=jnp.float32)

    @pl.when(i == 0)
    def _():
        sums_ref[...] = part

    @pl.when(i > 0)
    def _():
        sums_ref[...] += part

    @pl.when(i == _NBLK - 1)
    def _():
        s = sums_ref[...]
        g = s[:, :64] / jnp.maximum(s[:, 64:65], 1.0)
        h1 = jnp.maximum(
            jnp.dot(g, w1_ref[...].T, preferred_element_type=jnp.float32)
            + b1_ref[...], 0.0)
        out8 = jnp.dot(h1, w2_ref[...].T, preferred_element_type=jnp.float32)
        out_ref[...] = out8[:, 0:1] + b2_ref[0, 0]


def _run_post(ssum, acc01, acc23, batch3d, w1, b1, w2, b2):
    f32 = jnp.float32
    full = lambda shape: pl.BlockSpec(shape, lambda i: tuple(0 for _ in shape))
    return pl.pallas_call(
        _post_body,
        grid=(_NBLK,),
        in_specs=[pl.BlockSpec((NC, _BLK, 16), lambda i: (0, i, 0)),
                  pl.BlockSpec((NC, _BLK, 16), lambda i: (0, i, 0)),
                  pl.BlockSpec((NC, _BLK, 16), lambda i: (0, i, 0)),
                  pl.BlockSpec((1, 1, _BLK), lambda i: (i, 0, 0)),
                  full((32, 64)), full((1, 32)), full((8, 32)), full((1, 1))],
        out_specs=pl.BlockSpec((NG, 1), lambda i: (0, 0)),
        out_shape=jax.ShapeDtypeStruct((NG, 1), f32),
        scratch_shapes=[pltpu.VMEM((NG, 65), f32)],
    )(ssum, acc01, acc23, batch3d, w1, b1, w2, b2)


def kernel(x_tx, x_addr, edge_t2a, edge_a2t, batch_tx,
           proj_tx_W, proj_tx_b, proj_addr_W, proj_addr_b,
           han_tx_W, han_tx_b, han_addr_W, han_addr_b,
           att_src_t2a, att_dst_t2a, att_src_a2t, att_dst_a2t,
           q_sem, k_lin_W, k_lin_b, cls1_W, cls1_b, cls2_W, cls2_b):
    f32 = jnp.float32
    w_tx = han_tx_W @ proj_tx_W
    b_tx = (han_tx_W @ proj_tx_b + han_tx_b)[None, :]
    w_ad = han_addr_W @ proj_addr_W
    b_ad = (han_addr_W @ proj_addr_b + han_addr_b)[None, :]
    blockdiag = jnp.kron(jnp.eye(H_HEADS, dtype=f32),
                         jnp.ones((D_HEAD, 1), f32))
    a_src_m = blockdiag * att_src_a2t.reshape(HID, 1)
    a_dst_m = blockdiag * att_dst_a2t.reshape(HID, 1)

    z01, z23, msrc, mdst = _run_pre(
        x_tx, x_addr, w_tx, b_tx, w_ad, b_ad, a_src_m, a_dst_m)

    npad = E_PAD - E_EDGES
    src_p = jnp.concatenate(
        [edge_a2t[0], jnp.zeros((npad,), jnp.int32)]).reshape(E_PAD // CHUNK,
                                                             CHUNK)
    dst_p = jnp.concatenate(
        [edge_a2t[1], jnp.full((npad,), N_NODE, jnp.int32)]).reshape(
            E_PAD // CHUNK, CHUNK)

    e_all, ssum = _run_pass1(src_p, dst_p, msrc, mdst)
    acc01 = _run_pass2(0, 1, src_p, dst_p, e_all, z01)
    acc23 = _run_pass2(2, 3, src_p, dst_p, e_all, z23)

    batch3d = batch_tx.reshape(_NBLK, 1, _BLK)
    w2p = jnp.concatenate([cls2_W, jnp.zeros((7, 32), f32)], axis=0)
    return _run_post(ssum, acc01, acc23, batch3d,
                     cls1_W, cls1_b[None, :], w2p, cls2_b[None, :])


# speedup vs baseline: 13.7692x; 2.0052x over previous
import functools

import jax
import jax.numpy as jnp
from jax import lax
from jax.experimental import pallas as pl
from jax.experimental.pallas import tpu as pltpu
from jax.experimental.pallas import tpu_sc as plsc

N_NODE = 50000
E_EDGES = 800000
H_HEADS = 4
D_HEAD = 16
HID = 64
NG = 256

NC = 2
NS = 16
NW = NC * NS
CHUNK = 128
GROUP = 512
CPG = GROUP // CHUNK
E_PAD = 819200
N_PAD = 50176
GROUPS_PER_TILE = E_PAD // (NW * GROUP)
ZERO_ITERS = N_PAD // GROUP
ZERO_LOOPS = -(-ZERO_ITERS // NS)

_BLK = 2000
_NBLK = N_NODE // _BLK


def _pre_body(xt_ref, xa_ref, wt_ref, bt_ref, wa_ref, ba_ref, asrc_ref,
              adst_ref, z0_ref, z1_ref, msrc_ref, mdst_ref, msum_ref,
              mxs_ref, mxd_ref):
    i = pl.program_id(0)
    zt = jnp.dot(xt_ref[...], wt_ref[...].T,
                 preferred_element_type=jnp.float32, precision=lax.Precision.HIGHEST) + bt_ref[...]
    za = jnp.dot(xa_ref[...], wa_ref[...].T,
                 preferred_element_type=jnp.float32, precision=lax.Precision.HIGHEST) + ba_ref[...]
    z0_ref[0] = za[:, 0:16]
    z0_ref[1] = za[:, 16:32]
    z1_ref[0] = za[:, 32:48]
    z1_ref[1] = za[:, 48:64]
    pad = jnp.zeros((za.shape[0], 12), jnp.float32)
    a_s = jnp.dot(za, asrc_ref[...], preferred_element_type=jnp.float32, precision=lax.Precision.HIGHEST)
    a_d = jnp.dot(zt, adst_ref[...], preferred_element_type=jnp.float32, precision=lax.Precision.HIGHEST)
    msrc_ref[...] = jnp.concatenate([a_s, pad], axis=1)
    mdst_ref[...] = jnp.concatenate([a_d, pad], axis=1)
    bs = jnp.max(a_s, axis=0, keepdims=True)
    bd = jnp.max(a_d, axis=0, keepdims=True)

    @pl.when(i == 0)
    def _():
        mxs_ref[...] = bs
        mxd_ref[...] = bd

    @pl.when(i > 0)
    def _():
        mxs_ref[...] = jnp.maximum(mxs_ref[...], bs)
        mxd_ref[...] = jnp.maximum(mxd_ref[...], bd)

    @pl.when(i == _NBLK - 1)
    def _():
        msum_ref[...] = jnp.concatenate(
            [mxs_ref[...] + mxd_ref[...], jnp.zeros((1, 12), jnp.float32)],
            axis=1)


def _run_pre(x_tx, x_addr, w_tx, b_tx, w_ad, b_ad, a_src_m, a_dst_m):
    f32 = jnp.float32
    row_spec = pl.BlockSpec((_BLK, 65), lambda i: (i, 0))
    full = lambda shape: pl.BlockSpec(shape, lambda i: tuple(0 for _ in shape))
    out16 = pl.BlockSpec((_BLK, 16), lambda i: (i, 0))
    zspec = pl.BlockSpec((2, _BLK, 16), lambda i: (0, i, 0))
    return pl.pallas_call(
        _pre_body,
        grid=(_NBLK,),
        in_specs=[row_spec, row_spec, full((64, 65)), full((1, 64)),
                  full((64, 65)), full((1, 64)), full((64, 4)), full((64, 4))],
        out_specs=[zspec, zspec, out16, out16,
                   pl.BlockSpec((1, 16), lambda i: (0, 0))],
        out_shape=[jax.ShapeDtypeStruct((2, N_NODE, 16), f32)] * 2
        + [jax.ShapeDtypeStruct((N_NODE, 16), f32),
           jax.ShapeDtypeStruct((N_PAD, 16), f32),
           jax.ShapeDtypeStruct((1, 16), f32)],
        scratch_shapes=[pltpu.VMEM((1, 4), f32), pltpu.VMEM((1, 4), f32)],
    )(x_tx, x_addr, w_tx, b_tx, w_ad, b_ad, a_src_m, a_dst_m)


def _copy_idx(src_ref, dst_ref):
    for j in range(CPG):
        for k in range(8):
            sl = pl.ds(k * 16, 16)
            dst_ref[j, sl] = src_ref[j, sl]


def _pass1_body(src_hbm, dst_hbm, msrc_hbm, mdst_hbm, msum_hbm, e_hbm,
                ssum_hbm,
                srcv0, srcv1, dstv0, dstv1, msv0, msv1, mdv0, mdv1,
                ev0, ev1, sc0, sc1, mxv, ssum_sh,
                si0, si1, sg0, sg1, ss0, ss1):
    cid = lax.axis_index("c")
    sid = lax.axis_index("s")
    w = cid * NS + sid
    srcv = (srcv0, srcv1)
    dstv = (dstv0, dstv1)
    msv = (msv0, msv1)
    mdv = (mdv0, mdv1)
    ev = (ev0, ev1)
    scx = (sc0, sc1)
    si = (si0, si1)
    sg = (sg0, sg1)
    ss = (ss0, ss1)

    pltpu.sync_copy(msum_hbm, mxv)

    @pl.loop(0, GROUP)
    def _zero_ev(r):
        ev0[pl.ds(r, 1), :] = jnp.zeros((1, 16), jnp.float32)

    @pl.loop(0, ZERO_LOOPS)
    def _zero_shared(g):
        idx = sid + g * NS

        @pl.when(idx < ZERO_ITERS)
        def _():
            pltpu.sync_copy(ev0, ssum_sh.at[pl.ds(idx * GROUP, GROUP)])

    plsc.subcore_barrier()

    def fire_idx(b, g):
        chunk0 = w * (GROUPS_PER_TILE * CPG) + g * CPG
        pltpu.async_copy(src_hbm.at[pl.ds(chunk0, CPG)], srcv[b], si[b])
        pltpu.async_copy(dst_hbm.at[pl.ds(chunk0, CPG)], dstv[b], si[b])

    def wait_idx(b):
        pltpu.make_async_copy(src_hbm.at[pl.ds(0, CPG)], srcv[b], si[b]).wait()
        pltpu.make_async_copy(dst_hbm.at[pl.ds(0, CPG)], dstv[b], si[b]).wait()

    def fire_gathers(b):
        for j in range(CPG):
            pltpu.async_copy(msrc_hbm.at[srcv[b].at[j]],
                             msv[b].at[pl.ds(j * CHUNK, CHUNK)], sg[b])
            pltpu.async_copy(mdst_hbm.at[dstv[b].at[j]],
                             mdv[b].at[pl.ds(j * CHUNK, CHUNK)], sg[b])

    def wait_gathers(b):
        for j in range(CPG):
            pltpu.make_async_copy(msrc_hbm.at[srcv[b].at[j]],
                                  msv[b].at[pl.ds(j * CHUNK, CHUNK)],
                                  sg[b]).wait()
            pltpu.make_async_copy(mdst_hbm.at[dstv[b].at[j]],
                                  mdv[b].at[pl.ds(j * CHUNK, CHUNK)],
                                  sg[b]).wait()

    def compute(b):
        mrow = mxv[pl.ds(0, 1), :]

        @pl.loop(0, GROUP, step=4)
        def _row(r):
            for k in range(4):
                sl = pl.ds(r + k, 1)
                s = msv[b][sl, :] + mdv[b][sl, :]
                ev[b][sl, :] = jnp.exp(jnp.maximum(s, 0.2 * s) - mrow)

    def step(b, g):
        nb = 1 - b

        @pl.when(g + 1 < GROUPS_PER_TILE)
        def _():
            fire_idx(nb, g + 1)

        wait_gathers(b)

        @pl.when(g + 1 < GROUPS_PER_TILE)
        def _():
            wait_idx(nb)
            fire_gathers(nb)

        compute(b)
        pltpu.async_copy(
            ev[b], e_hbm.at[pl.ds((w * GROUPS_PER_TILE + g) * GROUP, GROUP)],
            ss[b])
        for j in range(CPG):
            pltpu.sync_copy(ev[b].at[pl.ds(j * CHUNK, CHUNK)],
                            ssum_sh.at[dstv[b].at[j]], add=True)
        pltpu.make_async_copy(ev[b], e_hbm.at[pl.ds(0, GROUP)], ss[b]).wait()

    fire_idx(0, 0)
    wait_idx(0)
    fire_gathers(0)

    @pl.loop(0, GROUPS_PER_TILE)
    def _group(g):
        @pl.when(g % 2 == 0)
        def _():
            step(0, g)

        @pl.when(g % 2 == 1)
        def _():
            step(1, g)

    plsc.subcore_barrier()

    @pl.loop(0, ZERO_LOOPS)
    def _writeout(g):
        idx = sid + g * NS

        @pl.when(idx < ZERO_ITERS)
        def _():
            pltpu.sync_copy(ssum_sh.at[pl.ds(idx * GROUP, GROUP)],
                            ssum_hbm.at[cid, pl.ds(idx * GROUP, GROUP)])


def _run_pass1(src2d, dst2d, msrc, mdst, msum):
    f32 = jnp.float32
    mesh = plsc.VectorSubcoreMesh(core_axis_name="c", subcore_axis_name="s")
    return pl.kernel(
        _pass1_body,
        mesh=mesh,
        compiler_params=pltpu.CompilerParams(use_tc_tiling_on_sc=False),
        out_type=[jax.ShapeDtypeStruct((E_PAD, 16), f32),
                  jax.ShapeDtypeStruct((NC, N_PAD, 16), f32)],
        scratch_types=[pltpu.VMEM((CPG, CHUNK), jnp.int32)] * 4
        + [pltpu.VMEM((GROUP, 16), f32)] * 6
        + [pltpu.VMEM((CPG, CHUNK), jnp.int32)] * 2
        + [pltpu.VMEM((1, 16), f32)]
        + [pltpu.VMEM_SHARED((N_PAD, 16), f32)]
        + [pltpu.SemaphoreType.DMA] * 6,
    )(src2d, dst2d, msrc, mdst, msum)


P2_GROUPS = E_PAD // (NS * GROUP)


def _pass2_body(h0, h1, src_hbm, dst_hbm, e_hbm, z_hbm, acc_hbm,
                srcv0, srcv1, dstv0, dstv1, ev0, ev1, zv0, zv1, mv0, mv1,
                sc0, sc1, acc_sh, si0, si1, sg0, sg1, ss0, ss1):
    cid = lax.axis_index("c")
    sid = lax.axis_index("s")
    srcv = (srcv0, srcv1)
    dstv = (dstv0, dstv1)
    ev = (ev0, ev1)
    zv = (zv0, zv1)
    mv = (mv0, mv1)
    scx = (sc0, sc1)
    si = (si0, si1)
    sg = (sg0, sg1)
    ss = (ss0, ss1)

    @pl.loop(0, GROUP)
    def _zero_mv(r):
        mv0[pl.ds(r, 1), :] = jnp.zeros((1, 16), jnp.float32)

    @pl.loop(0, ZERO_LOOPS)
    def _zero_shared(g):
        idx = sid + g * NS

        @pl.when(idx < ZERO_ITERS)
        def _():
            pltpu.sync_copy(mv0, acc_sh.at[pl.ds(idx * GROUP, GROUP)])

    plsc.subcore_barrier()

    def fire_idx(b, g):
        chunk0 = sid * (P2_GROUPS * CPG) + g * CPG
        base_e = (sid * P2_GROUPS + g) * GROUP
        pltpu.async_copy(src_hbm.at[pl.ds(chunk0, CPG)], srcv[b], si[b])
        pltpu.async_copy(dst_hbm.at[pl.ds(chunk0, CPG)], dstv[b], si[b])
        pltpu.async_copy(e_hbm.at[pl.ds(base_e, GROUP)], ev[b], si[b])

    def wait_idx(b):
        pltpu.make_async_copy(src_hbm.at[pl.ds(0, CPG)], srcv[b], si[b]).wait()
        pltpu.make_async_copy(dst_hbm.at[pl.ds(0, CPG)], dstv[b], si[b]).wait()
        pltpu.make_async_copy(e_hbm.at[pl.ds(0, GROUP)], ev[b], si[b]).wait()

    def fire_gathers(b):
        for j in range(CPG):
            pltpu.async_copy(z_hbm.at[cid].at[srcv[b].at[j]],
                             zv[b].at[pl.ds(j * CHUNK, CHUNK)], sg[b])

    def wait_gathers(b):
        for j in range(CPG):
            pltpu.make_async_copy(z_hbm.at[cid].at[srcv[b].at[j]],
                                  zv[b].at[pl.ds(j * CHUNK, CHUNK)],
                                  sg[b]).wait()

    def compute(b):
        @pl.loop(0, GROUP, step=4)
        def _row(r):
            for k in range(4):
                sl = pl.ds(r + k, 1)
                e_row = ev[b][sl, :]
                ea = jnp.where(cid == 0, e_row[0, h0], e_row[0, h1])
                mv[b][sl, :] = zv[b][sl, :] * ea

    def step(b, g):
        nb = 1 - b

        @pl.when(g + 1 < P2_GROUPS)
        def _():
            fire_idx(nb, g + 1)

        wait_gathers(b)

        @pl.when(g + 1 < P2_GROUPS)
        def _():
            wait_idx(nb)
            fire_gathers(nb)

        compute(b)
        for j in range(CPG):
            pltpu.sync_copy(mv[b].at[pl.ds(j * CHUNK, CHUNK)],
                            acc_sh.at[dstv[b].at[j]], add=True)

    fire_idx(0, 0)
    wait_idx(0)
    fire_gathers(0)

    @pl.loop(0, P2_GROUPS)
    def _group(g):
        @pl.when(g % 2 == 0)
        def _():
            step(0, g)

        @pl.when(g % 2 == 1)
        def _():
            step(1, g)

    plsc.subcore_barrier()

    @pl.loop(0, ZERO_LOOPS)
    def _writeout(g):
        idx = sid + g * NS

        @pl.when(idx < ZERO_ITERS)
        def _():
            pltpu.sync_copy(acc_sh.at[pl.ds(idx * GROUP, GROUP)],
                            acc_hbm.at[cid, pl.ds(idx * GROUP, GROUP)])


def _run_pass2(h0, h1, src2d, dst2d, e_all, zpair):
    f32 = jnp.float32
    mesh = plsc.VectorSubcoreMesh(core_axis_name="c", subcore_axis_name="s")
    return pl.kernel(
        functools.partial(_pass2_body, h0, h1),
        mesh=mesh,
        compiler_params=pltpu.CompilerParams(use_tc_tiling_on_sc=False),
        out_type=jax.ShapeDtypeStruct((NC, N_PAD, 16), f32),
        scratch_types=[pltpu.VMEM((CPG, CHUNK), jnp.int32)] * 4
        + [pltpu.VMEM((GROUP, 16), f32)] * 6
        + [pltpu.VMEM((CPG, CHUNK), jnp.int32)] * 2
        + [pltpu.VMEM_SHARED((N_PAD, 16), f32)]
        + [pltpu.SemaphoreType.DMA] * 6,
    )(src2d, dst2d, e_all, zpair)


def _post_body(ssum_ref, acc01_ref, acc23_ref, bt_ref, w1_ref, b1_ref,
               w2_ref, b2_ref, out_ref, sums_ref):
    i = pl.program_id(0)
    ssum = ssum_ref[0] + ssum_ref[1]
    cols = []
    for pair, ref in ((0, acc01_ref), (1, acc23_ref)):
        for k in range(2):
            h = pair * 2 + k
            a = ref[k]
            denom = ssum[:, h:h + 1] + 1e-16
            cols.append(a / denom)
    out_blk = jnp.maximum(jnp.concatenate(cols, axis=1), 0.0)
    ids = bt_ref[0, 0, :]
    onehot = (ids[:, None] == lax.broadcasted_iota(jnp.int32, (_BLK, NG), 1))
    onehot = onehot.astype(jnp.float32)
    feat = jnp.concatenate(
        [out_blk, jnp.ones((_BLK, 1), jnp.float32)], axis=1)
    part = lax.dot_general(onehot, feat, (((0,), (0,)), ((), ())),
                           preferred_element_type=jnp.float32, precision=lax.Precision.HIGHEST)

    @pl.when(i == 0)
    def _():
        sums_ref[...] = part

    @pl.when(i > 0)
    def _():
        sums_ref[...] += part

    @pl.when(i == _NBLK - 1)
    def _():
        s = sums_ref[...]
        g = s[:, :64] / jnp.maximum(s[:, 64:65], 1.0)
        h1 = jnp.maximum(
            jnp.dot(g, w1_ref[...].T, preferred_element_type=jnp.float32, precision=lax.Precision.HIGHEST)
            + b1_ref[...], 0.0)
        out8 = jnp.dot(h1, w2_ref[...].T, preferred_element_type=jnp.float32, precision=lax.Precision.HIGHEST)
        out_ref[...] = out8[:, 0:1] + b2_ref[0, 0]


def _run_post(ssum, acc01, acc23, batch3d, w1, b1, w2, b2):
    f32 = jnp.float32
    full = lambda shape: pl.BlockSpec(shape, lambda i: tuple(0 for _ in shape))
    return pl.pallas_call(
        _post_body,
        grid=(_NBLK,),
        in_specs=[pl.BlockSpec((NC, _BLK, 16), lambda i: (0, i, 0)),
                  pl.BlockSpec((NC, _BLK, 16), lambda i: (0, i, 0)),
                  pl.BlockSpec((NC, _BLK, 16), lambda i: (0, i, 0)),
                  pl.BlockSpec((1, 1, _BLK), lambda i: (i, 0, 0)),
                  full((32, 64)), full((1, 32)), full((8, 32)), full((1, 1))],
        out_specs=pl.BlockSpec((NG, 1), lambda i: (0, 0)),
        out_shape=jax.ShapeDtypeStruct((NG, 1), f32),
        scratch_shapes=[pltpu.VMEM((NG, 65), f32)],
    )(ssum, acc01, acc23, batch3d, w1, b1, w2, b2)


def kernel(x_tx, x_addr, edge_t2a, edge_a2t, batch_tx,
           proj_tx_W, proj_tx_b, proj_addr_W, proj_addr_b,
           han_tx_W, han_tx_b, han_addr_W, han_addr_b,
           att_src_t2a, att_dst_t2a, att_src_a2t, att_dst_a2t,
           q_sem, k_lin_W, k_lin_b, cls1_W, cls1_b, cls2_W, cls2_b):
    f32 = jnp.float32
    w_tx = han_tx_W @ proj_tx_W
    b_tx = (han_tx_W @ proj_tx_b + han_tx_b)[None, :]
    w_ad = han_addr_W @ proj_addr_W
    b_ad = (han_addr_W @ proj_addr_b + han_addr_b)[None, :]
    blockdiag = jnp.kron(jnp.eye(H_HEADS, dtype=f32),
                         jnp.ones((D_HEAD, 1), f32))
    a_src_m = blockdiag * att_src_a2t.reshape(HID, 1)
    a_dst_m = blockdiag * att_dst_a2t.reshape(HID, 1)

    z01, z23, msrc, mdst, msum = _run_pre(
        x_tx, x_addr, w_tx, b_tx, w_ad, b_ad, a_src_m, a_dst_m)

    npad = E_PAD - E_EDGES
    src_p = jnp.concatenate(
        [edge_a2t[0], jnp.zeros((npad,), jnp.int32)]).reshape(E_PAD // CHUNK,
                                                             CHUNK)
    dst_p = jnp.concatenate(
        [edge_a2t[1], jnp.full((npad,), N_NODE, jnp.int32)]).reshape(
            E_PAD // CHUNK, CHUNK)

    e_all, ssum = _run_pass1(src_p, dst_p, msrc, mdst, msum)
    acc01 = _run_pass2(0, 1, src_p, dst_p, e_all, z01)
    acc23 = _run_pass2(2, 3, src_p, dst_p, e_all, z23)

    batch3d = batch_tx.reshape(_NBLK, 1, _BLK)
    w2p = jnp.concatenate([cls2_W, jnp.zeros((7, 32), f32)], axis=0)
    return _run_post(ssum, acc01, acc23, batch3d,
                     cls1_W, cls1_b[None, :], w2p, cls2_b[None, :])
